# baseline (speedup 1.0000x reference)
import jax
import jax.numpy as jnp
from jax import lax
from jax.experimental import pallas as pl
from jax.experimental.pallas import tpu as pltpu
from jax.experimental.pallas import tpu_sc as plsc

B, T, C = 4, 4096, 2048
E = 16
TOP_K = 2
BM = 1024
N_TEC = 32
TPT = (B * T) // N_TEC


def _logits_block(x_ref, w_ref, out_ref):
    out_ref[...] = jnp.dot(
        x_ref[...], w_ref[...], preferred_element_type=jnp.float32)


@pl.kernel(
    out_type=[
        jax.ShapeDtypeStruct((B * T, E), jnp.float32),
        jax.ShapeDtypeStruct((B * T, E), jnp.int32),
    ],
    mesh=plsc.VectorSubcoreMesh(core_axis_name="c", subcore_axis_name="s"),
    scratch_types=[
        pltpu.VMEM((TPT, E), jnp.float32),
        pltpu.VMEM((TPT, E), jnp.float32),
        pltpu.VMEM((TPT, E), jnp.int32),
        pltpu.SemaphoreType.DMA,
        pltpu.SemaphoreType.DMA,
        pltpu.SemaphoreType.DMA,
    ],
    compiler_params=pltpu.CompilerParams(
        needs_layout_passes=False, use_tc_tiling_on_sc=False),
)
def _sc_router(logits_hbm, out_hbm, idx_hbm, lg_v, out_v, idx_v,
               sem_in, sem_out, sem_idx):
    c = lax.axis_index("c")
    s = lax.axis_index("s")
    base = (c * 16 + s) * TPT
    cp_in = pltpu.make_async_copy(
        logits_hbm.at[pl.ds(base, TPT), :], lg_v, sem_in)
    cp_in.start()
    cp_in.wait()

    def body(t, carry):
        v = lg_v[t, :]
        iota = lax.iota(jnp.int32, 16)
        m1 = jnp.max(v)
        i1 = jnp.min(jnp.where(v == m1, iota, E))
        masked = jnp.where(iota == i1, -jnp.inf, v)
        m2 = jnp.max(masked)
        i2 = jnp.min(jnp.where(masked == m2, iota, E))
        keep = (iota == i1) | (iota == i2)
        p = jnp.where(keep, jnp.exp(v - m1), 0.0)
        out_v[t, :] = p / jnp.sum(p)
        idx_v[t, :] = jnp.where(iota == 0, i1, i2)
        return carry

    lax.fori_loop(0, TPT, body, 0)

    cp_out = pltpu.make_async_copy(
        out_v, out_hbm.at[pl.ds(base, TPT), :], sem_out)
    cp_out.start()
    cp_idx = pltpu.make_async_copy(
        idx_v, idx_hbm.at[pl.ds(base, TPT), :], sem_idx)
    cp_idx.start()
    cp_out.wait()
    cp_idx.wait()


def kernel(x, W_route, W_noise):
    del W_noise
    xf = x.reshape(B * T, C)
    wT = W_route.T
    grid = (B * T // BM,)
    logits = pl.pallas_call(
        _logits_block,
        grid=grid,
        in_specs=[
            pl.BlockSpec((BM, C), lambda i: (i, 0)),
            pl.BlockSpec((C, E), lambda i: (0, 0)),
        ],
        out_specs=pl.BlockSpec((BM, E), lambda i: (i, 0)),
        out_shape=jax.ShapeDtypeStruct((B * T, E), jnp.float32),
        compiler_params=pltpu.CompilerParams(
            dimension_semantics=("parallel",),
        ),
    )(xf, wT)
    router_full, idx_wide = _sc_router(logits)
    router = router_full.reshape(B, T, E)
    indices = idx_wide[:, :TOP_K].reshape(B, T, TOP_K)
    return router, indices


# speedup vs baseline: 1.8095x; 1.8095x over previous
import jax
import jax.numpy as jnp
from jax.experimental import pallas as pl
from jax.experimental.pallas import tpu as pltpu

B, T, C = 4, 4096, 2048
E = 16
TOP_K = 2
BM = 1024
BM2 = BM // 2
IDX_ROWS = 8


def _router_block(xa_ref, xb_ref, w_ref, out_ref, idx_ref):
    dn = (((1,), (1,)), ((), ()))
    la = jax.lax.dot_general(
        w_ref[...], xa_ref[...], dn, preferred_element_type=jnp.float32)
    lb = jax.lax.dot_general(
        w_ref[...], xb_ref[...], dn, preferred_element_type=jnp.float32)
    logits = jnp.concatenate([la, lb], axis=1)
    iota = jax.lax.broadcasted_iota(jnp.int32, (E, BM), 0)

    m1 = jnp.max(logits, axis=0, keepdims=True)
    i1 = jnp.min(jnp.where(logits == m1, iota, E), axis=0, keepdims=True)
    masked = jnp.where(iota == i1, -jnp.inf, logits)
    m2 = jnp.max(masked, axis=0, keepdims=True)
    i2 = jnp.min(jnp.where(masked == m2, iota, E), axis=0, keepdims=True)

    keep = (iota == i1) | (iota == i2)
    p = jnp.where(keep, jnp.exp(logits - m1), 0.0)
    out_ref[...] = (p * (1.0 / (1.0 + jnp.exp(m2 - m1)))).T
    pair = jnp.concatenate([i1, i2], axis=0)
    idx_ref[...] = jnp.concatenate([pair, pair, pair, pair], axis=0)


def kernel(x, W_route, W_noise):
    del W_noise
    xf = x.reshape(B * T, C)
    grid = (B * T // BM,)
    outT, idxT = pl.pallas_call(
        _router_block,
        grid=grid,
        in_specs=[
            pl.BlockSpec((BM2, C), lambda i: (2 * i, 0)),
            pl.BlockSpec((BM2, C), lambda i: (2 * i + 1, 0)),
            pl.BlockSpec((E, C), lambda i: (0, 0)),
        ],
        out_specs=[
            pl.BlockSpec((BM, E), lambda i: (i, 0)),
            pl.BlockSpec((IDX_ROWS, BM), lambda i: (0, i)),
        ],
        out_shape=[
            jax.ShapeDtypeStruct((B * T, E), jnp.float32),
            jax.ShapeDtypeStruct((IDX_ROWS, B * T), jnp.int32),
        ],
        compiler_params=pltpu.CompilerParams(
            dimension_semantics=("parallel",),
        ),
    )(xf, xf, W_route)
    router = outT.reshape(B, T, E)
    indices = idxT[:TOP_K].T.reshape(B, T, TOP_K)
    return router, indices
